# baseline (speedup 1.0000x reference)
import functools

import jax
import jax.numpy as jnp
from jax import lax
from jax.experimental import pallas as pl
from jax.experimental.pallas import tpu as pltpu
from jax.experimental.pallas import tpu_sc as plsc

N = 10000
D = 128
H1 = 64
H2 = 32
HW = 32

NC = 2
NS = 16
CH = 128
NCH = 160
E_PAD = NS * NCH * CH
N_PAD = 10112
RPT = N_PAD // NS
TAIL = N - (NS - 1) * RPT
NBUF = 8
LAG = NBUF // 2


def _sc_body(table, src_idx, dst_idx, zeros, out,
             idx_s, idx_d, rows, acc, tbl, gsem, ssem):
    c = lax.axis_index("c")
    s = lax.axis_index("s")
    r0 = s * RPT

    pltpu.sync_copy(zeros.at[pl.ds(r0, RPT)], acc.at[pl.ds(r0, RPT)])
    @pl.when(s < NS - 1)
    def _stage_main():
        pltpu.sync_copy(table.at[c, pl.ds(r0, RPT)],
                        tbl.at[pl.ds(r0, RPT)])

    @pl.when(s == NS - 1)
    def _stage_tail():
        pltpu.sync_copy(table.at[c, pl.ds((NS - 1) * RPT, TAIL)],
                        tbl.at[pl.ds((NS - 1) * RPT, TAIL)])

    pltpu.sync_copy(src_idx.at[pl.ds(s * NCH, NCH)], idx_s)
    pltpu.sync_copy(dst_idx.at[pl.ds(s * NCH, NCH)], idx_d)
    plsc.subcore_barrier()

    for t in range(LAG):
        pltpu.async_copy(tbl.at[idx_s.at[t]], rows.at[t], gsem)

    def chunk(j, carry):
        b = lax.rem(j, NBUF)
        bnext = lax.rem(j + LAG, NBUF)

        @pl.when(j >= LAG)
        def _drain_lagged_scatter():
            pltpu.make_async_copy(
                rows.at[bnext], acc.at[idx_d.at[j - LAG]], ssem).wait()

        @pl.when(j + LAG < NCH)
        def _fire_ahead_gather():
            pltpu.async_copy(tbl.at[idx_s.at[j + LAG]], rows.at[bnext], gsem)

        pltpu.make_async_copy(tbl.at[idx_s.at[j]], rows.at[b], gsem).wait()
        pltpu.async_copy(rows.at[b], acc.at[idx_d.at[j]], ssem, add=True)
        return carry

    lax.fori_loop(0, NCH, chunk, 0)
    for t in range(max(NCH - LAG, 0), NCH):
        pltpu.make_async_copy(
            rows.at[t % NBUF], acc.at[idx_d.at[t]], ssem).wait()
    plsc.subcore_barrier()
    pltpu.sync_copy(acc.at[pl.ds(r0, RPT)], out.at[c, pl.ds(r0, RPT)])


_sc_scatter = functools.partial(
    pl.kernel,
    out_type=jax.ShapeDtypeStruct((NC, N_PAD, HW), jnp.float32),
    mesh=plsc.VectorSubcoreMesh(core_axis_name="c", subcore_axis_name="s"),
    scratch_types=[
        pltpu.VMEM((NCH, CH), jnp.int32),
        pltpu.VMEM((NCH, CH), jnp.int32),
        pltpu.VMEM((NBUF, CH, HW), jnp.float32),
        pltpu.VMEM_SHARED((N_PAD, HW), jnp.float32),
        pltpu.VMEM_SHARED((N, HW), jnp.float32),
        pltpu.SemaphoreType.DMA,
        pltpu.SemaphoreType.DMA,
    ],
    compiler_params=pltpu.CompilerParams(use_tc_tiling_on_sc=False),
)(_sc_body)


BM = 2000


def _mm_body(x_ref, w_ref, o_ref):
    s = jnp.dot(x_ref[...], w_ref[...], preferred_element_type=jnp.float32)
    o_ref[0] = s[:, :HW]
    o_ref[1] = s[:, HW:]


def _first_matmul(x, w1):
    return pl.pallas_call(
        _mm_body,
        grid=(N // BM,),
        in_specs=[
            pl.BlockSpec((BM, D), lambda i: (i, 0)),
            pl.BlockSpec((D, H1), lambda i: (0, 0)),
        ],
        out_specs=pl.BlockSpec((NC, BM, HW), lambda i: (0, i, 0)),
        out_shape=jax.ShapeDtypeStruct((NC, N_PAD, HW), jnp.float32),
    )(x, w1)


def _mid_body(h_ref, w_ref, o_ref):
    h = jnp.concatenate([h_ref[0], h_ref[1]], axis=1)
    s = jnp.dot(h, w_ref[...], preferred_element_type=jnp.float32)
    o_ref[0] = s[:, :HW]
    o_ref[1] = s[:, HW:]


def _mid_matmul(h1, w23):
    return pl.pallas_call(
        _mid_body,
        grid=(N // BM,),
        in_specs=[
            pl.BlockSpec((NC, BM, HW), lambda i: (0, i, 0)),
            pl.BlockSpec((H1, H1), lambda i: (0, 0)),
        ],
        out_specs=pl.BlockSpec((NC, BM, HW), lambda i: (0, i, 0)),
        out_shape=jax.ShapeDtypeStruct((NC, N_PAD, HW), jnp.float32),
    )(h1, w23)


def _final_body(g_ref, eps_ref, z_ref, mu_ref, lv_ref):
    mu = g_ref[0]
    lv = g_ref[1]
    mu_ref[...] = mu
    lv_ref[...] = lv
    z_ref[...] = eps_ref[...] * jnp.exp(lv) + mu


def _final_stage(g, eps):
    shp = jax.ShapeDtypeStruct((N, H2), jnp.float32)
    return pl.pallas_call(
        _final_body,
        grid=(N // BM,),
        in_specs=[
            pl.BlockSpec((NC, BM, HW), lambda i: (0, i, 0)),
            pl.BlockSpec((BM, H2), lambda i: (i, 0)),
        ],
        out_specs=[
            pl.BlockSpec((BM, H2), lambda i: (i, 0)),
            pl.BlockSpec((BM, H2), lambda i: (i, 0)),
            pl.BlockSpec((BM, H2), lambda i: (i, 0)),
        ],
        out_shape=[shp, shp, shp],
    )(g, eps)


def kernel(x, edge_index, W1, W2, W3, eps):
    pad = E_PAD - edge_index.shape[1]
    src = jnp.concatenate(
        [edge_index[0], jnp.zeros((pad,), jnp.int32)]).reshape(NS * NCH, CH)
    dst = jnp.concatenate(
        [edge_index[1], jnp.full((pad,), N, jnp.int32)]).reshape(NS * NCH, CH)
    zeros = jnp.zeros((N_PAD, HW), jnp.float32)
    w23 = jnp.concatenate([W2, W3], axis=1)

    s1 = _first_matmul(x, W1)
    h1 = _sc_scatter(s1, src, dst, zeros)
    s2 = _mid_matmul(h1, w23)
    g = _sc_scatter(s2, src, dst, zeros)
    z, mu, logvar = _final_stage(g, eps)
    return (z, mu, logvar)


# speedup vs baseline: 1.0038x; 1.0038x over previous
import functools

import jax
import jax.numpy as jnp
from jax import lax
from jax.experimental import pallas as pl
from jax.experimental.pallas import tpu as pltpu
from jax.experimental.pallas import tpu_sc as plsc

N = 10000
D = 128
H1 = 64
H2 = 32
HW = 32

NC = 2
NS = 16
CH = 128
NCH = 160
E_PAD = NS * NCH * CH
N_PAD = 10112
RPT = N_PAD // NS
TAIL = N - (NS - 1) * RPT
NBUF = 8
LAG = NBUF // 2


def _sc_body(table, src_idx, dst_idx, zeros, out,
             idx_s, idx_d, rows, acc, tbl, gsem, ssem):
    c = lax.axis_index("c")
    s = lax.axis_index("s")
    r0 = s * RPT

    pltpu.sync_copy(zeros.at[pl.ds(r0, RPT)], acc.at[pl.ds(r0, RPT)])
    @pl.when(s < NS - 1)
    def _stage_main():
        pltpu.sync_copy(table.at[c, pl.ds(r0, RPT)],
                        tbl.at[pl.ds(r0, RPT)])

    @pl.when(s == NS - 1)
    def _stage_tail():
        pltpu.sync_copy(table.at[c, pl.ds((NS - 1) * RPT, TAIL)],
                        tbl.at[pl.ds((NS - 1) * RPT, TAIL)])

    pltpu.sync_copy(src_idx.at[pl.ds(s * NCH, NCH)], idx_s)
    pltpu.sync_copy(dst_idx.at[pl.ds(s * NCH, NCH)], idx_d)
    plsc.subcore_barrier()

    for t in range(LAG):
        pltpu.async_copy(tbl.at[idx_s.at[t]], rows.at[t], gsem)

    def chunk(j, carry):
        b = lax.rem(j, NBUF)
        bnext = lax.rem(j + LAG, NBUF)

        @pl.when(j >= LAG)
        def _drain_lagged_scatter():
            pltpu.make_async_copy(
                rows.at[bnext], acc.at[idx_d.at[j - LAG]], ssem).wait()

        @pl.when(j + LAG < NCH)
        def _fire_ahead_gather():
            pltpu.async_copy(tbl.at[idx_s.at[j + LAG]], rows.at[bnext], gsem)

        pltpu.make_async_copy(tbl.at[idx_s.at[j]], rows.at[b], gsem).wait()
        pltpu.async_copy(rows.at[b], acc.at[idx_d.at[j]], ssem, add=True)
        return carry

    lax.fori_loop(0, NCH, chunk, 0)
    for t in range(max(NCH - LAG, 0), NCH):
        pltpu.make_async_copy(
            rows.at[t % NBUF], acc.at[idx_d.at[t]], ssem).wait()
    plsc.subcore_barrier()
    pltpu.sync_copy(acc.at[pl.ds(r0, RPT)], out.at[c, pl.ds(r0, RPT)])


_sc_scatter = functools.partial(
    pl.kernel,
    out_type=jax.ShapeDtypeStruct((NC, N_PAD, HW), jnp.float32),
    mesh=plsc.VectorSubcoreMesh(core_axis_name="c", subcore_axis_name="s"),
    scratch_types=[
        pltpu.VMEM((NCH, CH), jnp.int32),
        pltpu.VMEM((NCH, CH), jnp.int32),
        pltpu.VMEM((NBUF, CH, HW), jnp.float32),
        pltpu.VMEM_SHARED((N_PAD, HW), jnp.float32),
        pltpu.VMEM_SHARED((N, HW), jnp.float32),
        pltpu.SemaphoreType.DMA,
        pltpu.SemaphoreType.DMA,
    ],
    compiler_params=pltpu.CompilerParams(use_tc_tiling_on_sc=False),
)(_sc_body)


BM = 10000


def _mm_body(x_ref, w_ref, o_ref):
    s = jnp.dot(x_ref[...], w_ref[...], preferred_element_type=jnp.float32)
    o_ref[0] = s[:, :HW]
    o_ref[1] = s[:, HW:]


def _first_matmul(x, w1):
    return pl.pallas_call(
        _mm_body,
        grid=(N // BM,),
        in_specs=[
            pl.BlockSpec((BM, D), lambda i: (i, 0)),
            pl.BlockSpec((D, H1), lambda i: (0, 0)),
        ],
        out_specs=pl.BlockSpec((NC, BM, HW), lambda i: (0, i, 0)),
        out_shape=jax.ShapeDtypeStruct((NC, N_PAD, HW), jnp.float32),
    )(x, w1)


def _mid_body(h_ref, w_ref, o_ref):
    h = jnp.concatenate([h_ref[0], h_ref[1]], axis=1)
    s = jnp.dot(h, w_ref[...], preferred_element_type=jnp.float32)
    o_ref[0] = s[:, :HW]
    o_ref[1] = s[:, HW:]


def _mid_matmul(h1, w23):
    return pl.pallas_call(
        _mid_body,
        grid=(N // BM,),
        in_specs=[
            pl.BlockSpec((NC, BM, HW), lambda i: (0, i, 0)),
            pl.BlockSpec((H1, H1), lambda i: (0, 0)),
        ],
        out_specs=pl.BlockSpec((NC, BM, HW), lambda i: (0, i, 0)),
        out_shape=jax.ShapeDtypeStruct((NC, N_PAD, HW), jnp.float32),
    )(h1, w23)


def _final_body(g_ref, eps_ref, z_ref, mu_ref, lv_ref):
    mu = g_ref[0]
    lv = g_ref[1]
    mu_ref[...] = mu
    lv_ref[...] = lv
    z_ref[...] = eps_ref[...] * jnp.exp(lv) + mu


def _final_stage(g, eps):
    shp = jax.ShapeDtypeStruct((N, H2), jnp.float32)
    return pl.pallas_call(
        _final_body,
        grid=(N // BM,),
        in_specs=[
            pl.BlockSpec((NC, BM, HW), lambda i: (0, i, 0)),
            pl.BlockSpec((BM, H2), lambda i: (i, 0)),
        ],
        out_specs=[
            pl.BlockSpec((BM, H2), lambda i: (i, 0)),
            pl.BlockSpec((BM, H2), lambda i: (i, 0)),
            pl.BlockSpec((BM, H2), lambda i: (i, 0)),
        ],
        out_shape=[shp, shp, shp],
    )(g, eps)


def kernel(x, edge_index, W1, W2, W3, eps):
    pad = E_PAD - edge_index.shape[1]
    src = jnp.concatenate(
        [edge_index[0], jnp.zeros((pad,), jnp.int32)]).reshape(NS * NCH, CH)
    dst = jnp.concatenate(
        [edge_index[1], jnp.full((pad,), N, jnp.int32)]).reshape(NS * NCH, CH)
    zeros = jnp.zeros((N_PAD, HW), jnp.float32)
    w23 = jnp.concatenate([W2, W3], axis=1)

    s1 = _first_matmul(x, W1)
    h1 = _sc_scatter(s1, src, dst, zeros)
    s2 = _mid_matmul(h1, w23)
    g = _sc_scatter(s2, src, dst, zeros)
    z, mu, logvar = _final_stage(g, eps)
    return (z, mu, logvar)
